# baseline (speedup 1.0000x reference)
import functools

import jax
import jax.numpy as jnp
from jax import lax
from jax.experimental import pallas as pl
from jax.experimental.pallas import tpu as pltpu
from jax.experimental.pallas import tpu_sc as plsc


def _quadpack_body(x_ref, o_ref):
    x = x_ref[...]
    xt = x.T
    y = xt.reshape(xt.shape[0] // 4, 4, 32)
    o_ref[...] = jnp.concatenate([y[:, s, :] for s in range(4)], axis=1)


def _tc_quadpack(tT, C=2048):
    N = tT.shape[1]
    return pl.pallas_call(
        _quadpack_body,
        grid=((N + C - 1) // C,),
        in_specs=[pl.BlockSpec((32, C), lambda i: (0, i))],
        out_specs=pl.BlockSpec((C // 4, 128), lambda i: (i, 0)),
        out_shape=jax.ShapeDtypeStruct((N // 4, 128), jnp.float32),
    )(tT)


def _sc_gather(idxr_u, idxr_b, tbl_u, tbl_b):
    B = idxr_u.shape[0]
    info = plsc.get_sparse_core_info()
    NW = info.num_cores * info.num_subcores
    bpw = B // NW
    nch = bpw // 128
    mesh = plsc.VectorSubcoreMesh(core_axis_name="c", subcore_axis_name="s")

    @functools.partial(
        pl.kernel,
        mesh=mesh,
        out_type=(
            jax.ShapeDtypeStruct((B, 128), jnp.float32),
            jax.ShapeDtypeStruct((B, 128), jnp.float32),
        ),
        scratch_types=[
            pltpu.VMEM((nch, 8, 128), jnp.int32),
            pltpu.VMEM((bpw // 2, 128), jnp.float32),
            pltpu.VMEM((nch, 8, 128), jnp.int32),
            pltpu.VMEM((bpw // 2, 128), jnp.float32),
            pltpu.SemaphoreType.DMA,
        ],
    )
    def gk(iu_h, ib_h, tu_h, tb_h, u4_o, b4_o, idxu, rowsu, idxb, rowsb, sem):
        wid = lax.axis_index("s") * info.num_cores + lax.axis_index("c")
        base = pl.multiple_of(wid * bpw, bpw)
        for r in range(nch):
            pltpu.sync_copy(iu_h.at[pl.ds(base + r * 128, 128)], idxu.at[r, 0])
            pltpu.sync_copy(ib_h.at[pl.ds(base + r * 128, 128)], idxb.at[r, 0])
        for half in range(2):
            cps = []
            for r in range(nch // 2):
                rr = half * (nch // 2) + r
                cps.append(pltpu.async_copy(
                    tu_h.at[idxu.at[rr, 0]],
                    rowsu.at[pl.ds(r * 128, 128)], sem))
                cps.append(pltpu.async_copy(
                    tb_h.at[idxb.at[rr, 0]],
                    rowsb.at[pl.ds(r * 128, 128)], sem))
            for cp in cps:
                cp.wait()
            hb = base + half * (bpw // 2)
            pltpu.sync_copy(rowsu, u4_o.at[pl.ds(hb, bpw // 2)])
            pltpu.sync_copy(rowsb, b4_o.at[pl.ds(hb, bpw // 2)])

    return gk(idxr_u, idxr_b, tbl_u, tbl_b)


def _mlp_body(u4, b4, qu, qb, gu, gb, b1, w2, b2, wo, bo, out):
    x_u = u4[...]
    x_b = b4[...]
    h = b1[...] + jnp.zeros_like(x_u[:, :1])
    for q in range(4):
        mu = (qu[...] == q).astype(jnp.float32)
        mb = (qb[...] == q).astype(jnp.float32)
        h = h + mu * jnp.dot(x_u, gu[q * 128:(q + 1) * 128, :],
                             preferred_element_type=jnp.float32)
        h = h + mb * jnp.dot(x_b, gb[q * 128:(q + 1) * 128, :],
                             preferred_element_type=jnp.float32)
    h = jnp.maximum(h, 0.0)
    h = jnp.dot(h, w2[...], preferred_element_type=jnp.float32)
    h = jnp.maximum(h + b2[...], 0.0)
    o = jnp.sum(h * wo[...], axis=1, keepdims=True) + bo[...]
    out[...] = jax.nn.sigmoid(o)


def _tc_mlp(u4, b4, qu, qb, Gu, Gb, b1, W2, b2, Wout, bout, blk=2048):
    B = u4.shape[0]
    H1 = W2.shape[1]
    H2 = W2.shape[0]
    w2 = W2.T
    full = lambda shape: pl.BlockSpec(shape, lambda i: (0, 0))
    return pl.pallas_call(
        _mlp_body,
        grid=(B // blk,),
        in_specs=[
            pl.BlockSpec((blk, 128), lambda i: (i, 0)),
            pl.BlockSpec((blk, 128), lambda i: (i, 0)),
            pl.BlockSpec((blk, 1), lambda i: (i, 0)),
            pl.BlockSpec((blk, 1), lambda i: (i, 0)),
            full((512, H1)),
            full((512, H1)),
            full((1, H1)),
            full((H1, H2)),
            full((1, H2)),
            full((1, H2)),
            full((1, 1)),
        ],
        out_specs=pl.BlockSpec((blk, 1), lambda i: (i, 0)),
        out_shape=jax.ShapeDtypeStruct((B, 1), jnp.float32),
    )(u4, b4, qu, qb, Gu, Gb, b1.reshape(1, H1), w2, b2.reshape(1, H2),
      Wout, bout.reshape(1, 1))


def _shifted_w(w_half):
    H1 = w_half.shape[1]
    g = jnp.zeros((4, 4, 32, H1), jnp.float32)
    for q in range(4):
        g = g.at[q, q].set(w_half)
    return g.reshape(512, H1)


def kernel(user, book, user_table, book_table, W1, b1, W2, b2, Wout, bout):
    B = user.shape[0]
    E = user_table.shape[1]
    user = user.astype(jnp.int32)
    book = book.astype(jnp.int32)
    tbl_u = _tc_quadpack(user_table.T)
    tbl_b = _tc_quadpack(book_table.T)
    u4, b4 = _sc_gather(user // 4, book // 4, tbl_u, tbl_b)
    qu = (user % 4).reshape(B, 1)
    qb = (book % 4).reshape(B, 1)
    Gu = _shifted_w(W1[:, :E].T)
    Gb = _shifted_w(W1[:, E:].T)
    return _tc_mlp(u4, b4, qu, qb, Gu, Gb, b1, W2, b2, Wout, bout)


# speedup vs baseline: 2.4149x; 2.4149x over previous
import functools

import jax
import jax.numpy as jnp
from jax import lax
from jax.experimental import pallas as pl
from jax.experimental.pallas import tpu as pltpu
from jax.experimental.pallas import tpu_sc as plsc


_S = 256000
_QC = 2048


def _quadpack_body(x0, x1, x2, x3, eye_ref, o_ref, *, n):
    k = pl.program_id(0)
    col = 3 * _S + k * _QC + lax.broadcasted_iota(jnp.int32, (32, _QC), 1)
    x3v = jnp.where(col < n, x3[...], 0.0)
    xbig = jnp.concatenate([x0[...], x1[...], x2[...], x3v], axis=0)
    o_ref[...] = lax.dot_general(xbig, eye_ref[...], (((0,), (0,)), ((), ())),
                                 precision=lax.Precision.HIGHEST,
                                 preferred_element_type=jnp.float32)


def _tc_quadpack(tT):
    N = tT.shape[1]
    nb = _S // _QC
    maxb = (N - 1) // _QC
    spec = lambda s: pl.BlockSpec(
        (32, _QC), lambda k, s=s: (0, jnp.minimum(s * nb + k, maxb)))
    return pl.pallas_call(
        functools.partial(_quadpack_body, n=N),
        grid=(nb,),
        in_specs=[spec(0), spec(1), spec(2), spec(3),
                  pl.BlockSpec((128, 128), lambda k: (0, 0))],
        out_specs=pl.BlockSpec((_QC, 128), lambda k: (k, 0)),
        out_shape=jax.ShapeDtypeStruct((_S, 128), jnp.float32),
    )(tT, tT, tT, tT, jnp.eye(128, dtype=jnp.float32))


def _sc_gather(idxr_u, idxr_b, tbl_u, tbl_b):
    B = idxr_u.shape[0]
    info = plsc.get_sparse_core_info()
    NW = info.num_cores * info.num_subcores
    bpw = B // NW
    mesh = plsc.VectorSubcoreMesh(core_axis_name="c", subcore_axis_name="s")

    @functools.partial(
        pl.kernel,
        mesh=mesh,
        out_type=(
            jax.ShapeDtypeStruct((B, 128), jnp.float32),
            jax.ShapeDtypeStruct((B, 128), jnp.float32),
        ),
        scratch_types=[
            pltpu.VMEM((bpw,), jnp.int32),
            pltpu.VMEM((bpw, 128), jnp.float32),
            pltpu.SemaphoreType.DMA,
        ],
    )
    def gk(iu_h, ib_h, tu_h, tb_h, u4_o, b4_o, idx_v, rows_v, sem):
        wid = lax.axis_index("s") * info.num_cores + lax.axis_index("c")
        base = pl.multiple_of(wid * bpw, bpw)
        pltpu.sync_copy(iu_h.at[pl.ds(base, bpw)], idx_v)
        pltpu.async_copy(tu_h.at[idx_v], rows_v, sem).wait()
        pltpu.sync_copy(rows_v, u4_o.at[pl.ds(base, bpw)])
        pltpu.sync_copy(ib_h.at[pl.ds(base, bpw)], idx_v)
        pltpu.async_copy(tb_h.at[idx_v], rows_v, sem).wait()
        pltpu.sync_copy(rows_v, b4_o.at[pl.ds(base, bpw)])

    return gk(idxr_u, idxr_b, tbl_u, tbl_b)


def _mlp_body(u4, b4, qu, qb, gu, gb, b1, w2, b2, wo, bo, out):
    x_u = u4[...]
    x_b = b4[...]
    h = b1[...] + jnp.zeros_like(x_u[:, :1])
    for q in range(4):
        mu = (qu[...] == q).astype(jnp.float32)
        mb = (qb[...] == q).astype(jnp.float32)
        h = h + mu * jnp.dot(x_u, gu[q * 128:(q + 1) * 128, :],
                             preferred_element_type=jnp.float32)
        h = h + mb * jnp.dot(x_b, gb[q * 128:(q + 1) * 128, :],
                             preferred_element_type=jnp.float32)
    h = jnp.maximum(h, 0.0)
    h = jnp.dot(h, w2[...], preferred_element_type=jnp.float32)
    h = jnp.maximum(h + b2[...], 0.0)
    o = jnp.sum(h * wo[...], axis=1, keepdims=True) + bo[...]
    out[...] = jax.nn.sigmoid(o)


def _tc_mlp(u4, b4, qu, qb, Gu, Gb, b1, W2, b2, Wout, bout, blk=2048):
    B = u4.shape[0]
    H1 = W2.shape[1]
    H2 = W2.shape[0]
    w2 = W2.T
    full = lambda shape: pl.BlockSpec(shape, lambda i: (0, 0))
    return pl.pallas_call(
        _mlp_body,
        grid=(B // blk,),
        in_specs=[
            pl.BlockSpec((blk, 128), lambda i: (i, 0)),
            pl.BlockSpec((blk, 128), lambda i: (i, 0)),
            pl.BlockSpec((blk, 1), lambda i: (i, 0)),
            pl.BlockSpec((blk, 1), lambda i: (i, 0)),
            full((512, H1)),
            full((512, H1)),
            full((1, H1)),
            full((H1, H2)),
            full((1, H2)),
            full((1, H2)),
            full((1, 1)),
        ],
        out_specs=pl.BlockSpec((blk, 1), lambda i: (i, 0)),
        out_shape=jax.ShapeDtypeStruct((B, 1), jnp.float32),
    )(u4, b4, qu, qb, Gu, Gb, b1.reshape(1, H1), w2, b2.reshape(1, H2),
      Wout, bout.reshape(1, 1))


def _shifted_w(w_half):
    H1 = w_half.shape[1]
    g = jnp.zeros((4, 4, 32, H1), jnp.float32)
    for q in range(4):
        g = g.at[q, q].set(w_half)
    return g.reshape(512, H1)


def kernel(user, book, user_table, book_table, W1, b1, W2, b2, Wout, bout):
    B = user.shape[0]
    E = user_table.shape[1]
    user = user.astype(jnp.int32)
    book = book.astype(jnp.int32)
    tbl_u = _tc_quadpack(user_table.T)
    tbl_b = _tc_quadpack(book_table.T)
    u4, b4 = _sc_gather(user % _S, book % _S, tbl_u, tbl_b)
    qu = (user // _S).reshape(B, 1)
    qb = (book // _S).reshape(B, 1)
    Gu = _shifted_w(W1[:, :E].T)
    Gb = _shifted_w(W1[:, E:].T)
    return _tc_mlp(u4, b4, qu, qb, Gu, Gb, b1, W2, b2, Wout, bout)
